# baseline (speedup 1.0000x reference)
import functools

import jax
import jax.numpy as jnp
from jax import lax
from jax.experimental import pallas as pl
from jax.experimental.pallas import tpu as pltpu
from jax.experimental.pallas import tpu_sc as plsc

N = 10000
E = 320000
D = 128

NC = 2
NS = 16
NW = NC * NS
L = 16

RPT = 320
RACC = 328
NPAD = NW * RPT
EPW = 10240
EPAD = EPW * NW
C = 128
NCHW = EPW // C
MAXCHW = 112
GCH = NW * MAXCHW
NB = 4

_mesh = plsc.VectorSubcoreMesh(core_axis_name="c", subcore_axis_name="s")
_params = pltpu.CompilerParams(needs_layout_passes=False)

_II = None


def _iota16():
    return lax.iota(jnp.int32, L)


def _key_of(d):
    return ((d >> 6) * 13108) >> 16


def _extract_static(v0, v1, k):
    v = v0 if k < L else v1
    return jnp.sum(jnp.where(_iota16() == (k % L), v, 0))


def _extract_dyn(v0, v1, idx):
    ii = _iota16()
    lo = jnp.where(idx < L, idx, idx - L)
    s0 = jnp.where((ii == lo) & (idx < L), v0, 0)
    s1 = jnp.where((ii == lo) & (idx >= L), v1, 0)
    return jnp.sum(s0 + s1)


def _excl_cumsum_pair(x0, x1):
    c0 = plsc.cumsum(x0) - x0
    c1 = plsc.cumsum(x1) - x1 + jnp.sum(x0)
    return c0, c1


@functools.partial(
    pl.kernel,
    out_type=jax.ShapeDtypeStruct((NW, 2 * L), jnp.int32),
    mesh=_mesh,
    scratch_types=[
        pltpu.VMEM((2, EPW), jnp.int32),
        pltpu.VMEM((2 * L,), jnp.int32),
    ],
    compiler_params=_params,
)
def _sc_hist(idx_hbm, hist_out, idxv, histv):
    wid = lax.axis_index("c") * NS + lax.axis_index("s")
    pltpu.sync_copy(idx_hbm.at[wid], idxv)
    zeros = jnp.zeros((L,), jnp.int32)
    histv[pl.ds(0, L)] = zeros
    histv[pl.ds(L, L)] = zeros
    ones = jnp.ones((L,), jnp.int32)

    def body(i, carry):
        d = idxv[1, pl.ds(i * L, L)]
        plsc.addupdate_scatter(histv, [_key_of(d)], ones)
        return carry

    lax.fori_loop(0, EPW // L, body, 0)
    pltpu.sync_copy(histv, hist_out.at[wid])


@functools.partial(
    pl.kernel,
    out_type=jax.ShapeDtypeStruct((GCH, 2, C), jnp.int32),
    mesh=_mesh,
    scratch_types=[
        pltpu.VMEM((2, EPW), jnp.int32),
        pltpu.VMEM((NW, 2 * L), jnp.int32),
        pltpu.VMEM((2 * L,), jnp.int32),
        pltpu.VMEM((L, 2 * L), jnp.int32),
        pltpu.VMEM((MAXCHW, 2, C), jnp.int32),
        pltpu.SemaphoreType.DMA,
    ],
    compiler_params=_params,
)
def _sc_bucket(idx_hbm, hist_hbm, g3_out, idxv, histall, minev, cnt2, stg, osem):
    wid = lax.axis_index("c") * NS + lax.axis_index("s")
    pltpu.sync_copy(idx_hbm.at[wid], idxv)
    pltpu.sync_copy(hist_hbm, histall)
    pltpu.sync_copy(hist_hbm.at[wid], minev)
    ii = _iota16()

    tot0 = jnp.zeros((L,), jnp.int32)
    tot1 = jnp.zeros((L,), jnp.int32)
    part0 = jnp.zeros((L,), jnp.int32)
    part1 = jnp.zeros((L,), jnp.int32)
    for w in range(NW):
        h0 = histall[w, pl.ds(0, L)]
        h1 = histall[w, pl.ds(L, L)]
        m0 = (h0 + (C - 1)) >> 7
        m1 = (h1 + (C - 1)) >> 7
        tot0 = tot0 + m0
        tot1 = tot1 + m1
        before = jnp.int32(w) < wid
        part0 = part0 + jnp.where(before, m0, 0)
        part1 = part1 + jnp.where(before, m1, 0)
    pref0, pref1 = _excl_cumsum_pair(tot0, tot1)
    gs0 = pref0 + part0
    gs1 = pref1 + part1

    my0 = minev[pl.ds(0, L)]
    my1 = minev[pl.ds(L, L)]
    mn0 = (my0 + (C - 1)) >> 7
    mn1 = (my1 + (C - 1)) >> 7
    ls0, ls1 = _excl_cumsum_pair(mn0, mn1)

    zl = jnp.zeros((L,), jnp.int32)
    for l in range(L):
        cnt2[l, pl.ds(0, L)] = zl
        cnt2[l, pl.ds(L, L)] = zl
    ones = jnp.ones((L,), jnp.int32)

    def hbody(i, carry):
        d = idxv[1, pl.ds(i * L, L)]
        plsc.addupdate_scatter(cnt2, [ii, _key_of(d)], ones)
        return carry

    lax.fori_loop(0, EPW // L, hbody, 0)

    run0 = ls0 * C
    run1 = ls1 * C
    for l in range(L):
        r0 = cnt2[l, pl.ds(0, L)]
        r1 = cnt2[l, pl.ds(L, L)]
        cnt2[l, pl.ds(0, L)] = run0
        cnt2[l, pl.ds(L, L)] = run1
        run0 = run0 + r0
        run1 = run1 + r1

    for k in range(NW):
        lsk = _extract_static(ls0, ls1, k)
        nck = _extract_static(mn0, mn1, k)
        dumv = jnp.full((L,), RPT * k + RACC - 1, jnp.int32)
        zv = jnp.zeros((L,), jnp.int32)

        def fbody(t, carry, lsk=lsk, dumv=dumv, zv=zv):
            for j in range(C // L):
                stg[lsk + t, 0, pl.ds(j * L, L)] = zv
                stg[lsk + t, 1, pl.ds(j * L, L)] = dumv
            return carry

        lax.fori_loop(0, nck, fbody, 0)

    def pbody(i, carry):
        s = idxv[0, pl.ds(i * L, L)]
        d = idxv[1, pl.ds(i * L, L)]
        key = _key_of(d)
        pos = plsc.load_gather(cnt2, [ii, key])
        plsc.addupdate_scatter(cnt2, [ii, key], ones)
        ch = pos >> 7
        slot = pos & (C - 1)
        plsc.store_scatter(stg, [ch, jnp.zeros((L,), jnp.int32), slot], s)
        plsc.store_scatter(stg, [ch, jnp.ones((L,), jnp.int32), slot], d)
        return carry

    lax.fori_loop(0, EPW // L, pbody, 0)

    nissued = jnp.int32(0)
    for k in range(NW):
        lsk = _extract_static(ls0, ls1, k)
        nck = _extract_static(mn0, mn1, k)
        gsk = _extract_static(gs0, gs1, k)

        def obody(t, carry, lsk=lsk, gsk=gsk):
            pltpu.async_copy(stg.at[lsk + t], g3_out.at[gsk + t], osem)
            return carry

        lax.fori_loop(0, nck, obody, 0)
        nissued = nissued + nck

    def dbody(t, carry):
        pltpu.make_async_copy(stg.at[0], g3_out.at[0], osem).wait()
        return carry

    lax.fori_loop(0, nissued, dbody, 0)


def _make_sc_agg(with_deg):
    out_type = [jax.ShapeDtypeStruct((NW, RPT, D), jnp.float32)]
    if with_deg:
        out_type.append(jax.ShapeDtypeStruct((NW, RACC), jnp.float32))

    def body(g3_hbm, hist_hbm, feat_hbm, *rest):
        if with_deg:
            acc_out, deg_out, histall, accv, degv, idxc, rowsv, *sems = rest
        else:
            degv = None
            acc_out, histall, accv, idxc, rowsv, *sems = rest
        isem = sems[0:NB]
        gsem = sems[NB:2 * NB]
        o = lax.axis_index("c") * NS + lax.axis_index("s")
        pltpu.sync_copy(hist_hbm, histall)

        tot0 = jnp.zeros((L,), jnp.int32)
        tot1 = jnp.zeros((L,), jnp.int32)
        for w in range(NW):
            h0 = histall[w, pl.ds(0, L)]
            h1 = histall[w, pl.ds(L, L)]
            tot0 = tot0 + ((h0 + (C - 1)) >> 7)
            tot1 = tot1 + ((h1 + (C - 1)) >> 7)
        pref0, pref1 = _excl_cumsum_pair(tot0, tot1)
        gbase = _extract_dyn(pref0, pref1, o)
        nch = _extract_dyn(tot0, tot1, o)

        zf = jnp.zeros((L,), jnp.float32)

        def zbody(r, carry):
            for j in range(D // L):
                accv[r, pl.ds(j * L, L)] = zf
            return carry

        lax.fori_loop(0, RACC, zbody, 0)
        if with_deg:
            def zdbody(i, carry):
                degv[pl.ds(i * L, L)] = zf
                return carry
            lax.fori_loop(0, RACC // L, zdbody, 0)

        onesf = jnp.ones((L,), jnp.float32)
        base = o * RPT
        ii = _iota16()
        eidx = [ii + j * L for j in range(C // L)]

        for b in range(NB):
            @pl.when(b < nch)
            def _(b=b):
                pltpu.async_copy(g3_hbm.at[gbase + b], idxc.at[b], isem[b])

        nrounds = (nch + (NB - 1)) >> 2

        def body_k(k, carry):
            for b in range(NB):
                t = k * NB + b

                @pl.when(t < nch)
                def _(b=b, t=t):
                    pltpu.make_async_copy(
                        g3_hbm.at[gbase + t], idxc.at[b], isem[b]).wait()
                    pltpu.async_copy(
                        feat_hbm.at[idxc.at[b, 0]], rowsv.at[b], gsem[b])
            for b in range(NB):
                t = k * NB + b

                @pl.when(t < nch)
                def _(b=b, t=t):
                    lrows = []
                    for j in range(C // L):
                        d = idxc[b, 1, pl.ds(j * L, L)]
                        lrow = d - base
                        lrows.append(lrow)
                        if with_deg:
                            plsc.addupdate_scatter(degv, [lrow], onesf)
                    tn = t + NB

                    @pl.when(tn < nch)
                    def _():
                        pltpu.async_copy(
                            g3_hbm.at[gbase + tn], idxc.at[b], isem[b])
                    pltpu.make_async_copy(
                        feat_hbm.at[idxc.at[b, 0]], rowsv.at[b], gsem[b]).wait()

                    def cbody(c, carry2, b=b, lrows=lrows):
                        cs = jnp.zeros((L,), jnp.int32) + c
                        for j in range(C // L):
                            val = plsc.load_gather(
                                rowsv.at[b], [eidx[j], cs])
                            plsc.addupdate_scatter(
                                accv, [lrows[j], cs], val)
                        return carry2

                    lax.fori_loop(0, D, cbody, 0)
            return carry

        lax.fori_loop(0, nrounds, body_k, 0)

        pltpu.sync_copy(accv.at[pl.ds(0, RPT)], acc_out.at[o])
        if with_deg:
            pltpu.sync_copy(degv, deg_out.at[o])

    scratch = [
        pltpu.VMEM((NW, 2 * L), jnp.int32),
        pltpu.VMEM((RACC, D), jnp.float32),
    ]
    if with_deg:
        scratch.append(pltpu.VMEM((RACC,), jnp.float32))
    scratch += [
        pltpu.VMEM((NB, 2, C), jnp.int32),
        pltpu.VMEM((NB, C, D), jnp.float32),
    ]
    scratch += [pltpu.SemaphoreType.DMA] * (2 * NB)
    return pl.kernel(body, out_type=out_type, mesh=_mesh,
                     scratch_types=scratch, compiler_params=_params)


_sc_agg_deg = _make_sc_agg(True)
_sc_agg = _make_sc_agg(False)


def _bn_relu(z, gamma, beta, eps=1e-5):
    mu = jnp.mean(z, axis=0)
    var = jnp.mean(jnp.square(z - mu), axis=0)
    return jnp.maximum(gamma * (z - mu) * lax.rsqrt(var + eps) + beta, 0.0)


def _tc_stage1(acc_ref, deg_ref, w0, b0, g0, be0, w1, b1, g1, be1,
               s1pad_ref, t1_ref, invdeg_ref):
    deg = deg_ref[0:N, 0]
    inv_deg = (1.0 / jnp.clip(deg, 1.0, None))[:, None]
    a0 = acc_ref[0:N, :] * inv_deg
    z0 = jnp.dot(a0, w0[...], preferred_element_type=jnp.float32) + b0[...]
    s1 = _bn_relu(z0, g0[...], be0[...])
    z1 = jnp.dot(a0, w1[...], preferred_element_type=jnp.float32) + b1[...]
    t1_ref[...] = _bn_relu(z1, g1[...], be1[...])
    s1pad_ref[...] = jnp.concatenate(
        [s1, jnp.zeros((NPAD - N, D), jnp.float32)], axis=0)
    invdeg_ref[...] = inv_deg


def _tc_stage2(acc_ref, invdeg_ref, s1pad_ref, t1_ref, h_ref,
               w2, b2, g2, be2, wc, bc, gc, bec, out_ref):
    a1 = acc_ref[0:N, :] * invdeg_ref[...]
    z2 = jnp.dot(a1, w2[...], preferred_element_type=jnp.float32) + b2[...]
    s2 = t1_ref[...] + _bn_relu(z2, g2[...], be2[...])
    s1 = s1pad_ref[0:N, :]
    zc = (jnp.dot(s1, wc[0:D, :], preferred_element_type=jnp.float32)
          + jnp.dot(s2, wc[D:2 * D, :], preferred_element_type=jnp.float32)
          + bc[...])
    out_ref[...] = h_ref[...] + _bn_relu(zc, gc[...], bec[...])


def kernel(h, edge_index, W0, b0, g0, be0, W1, b1, g1, be1,
           W2, b2, g2, be2, Wc, bc, gc, bec):
    src = edge_index[0]
    dst = edge_index[1]
    pad = EPAD - E
    src_pad = jnp.concatenate([src, jnp.zeros((pad,), jnp.int32)])
    dst_pad = jnp.concatenate([dst, jnp.full((pad,), N, jnp.int32)])
    idx2 = jnp.stack([src_pad.reshape(NW, EPW),
                      dst_pad.reshape(NW, EPW)], axis=1)
    hpad = jnp.concatenate(
        [h, jnp.zeros((NPAD - N, D), jnp.float32)], axis=0)

    hist = _sc_hist(idx2)
    g3 = _sc_bucket(idx2, hist)

    acc_a, deg = _sc_agg_deg(g3, hist, hpad)
    acc_a = acc_a.reshape(NPAD, D)
    deg = deg[:, :RPT].reshape(NPAD, 1)

    s1pad, t1, inv_deg = pl.pallas_call(
        _tc_stage1,
        out_shape=[
            jax.ShapeDtypeStruct((NPAD, D), jnp.float32),
            jax.ShapeDtypeStruct((N, D), jnp.float32),
            jax.ShapeDtypeStruct((N, 1), jnp.float32),
        ],
    )(acc_a, deg, W0, b0, g0, be0, W1, b1, g1, be1)

    acc_b = _sc_agg(g3, hist, s1pad)[0].reshape(NPAD, D)

    out = pl.pallas_call(
        _tc_stage2,
        out_shape=jax.ShapeDtypeStruct((N, D), jnp.float32),
    )(acc_b, inv_deg, s1pad, t1, h, W2, b2, g2, be2, Wc, bc, gc, bec)
    return out
